# Initial kernel scaffold; baseline (speedup 1.0000x reference)
import jax
import jax.numpy as jnp
from jax.experimental import pallas as pl


def kernel(node_features, edge_index, edge_weights, W, b):
    raise NotImplementedError("write your pallas kernel here")


# speedup vs baseline: 5.1326x; 5.1326x over previous
import functools

import jax
import jax.numpy as jnp
from jax import lax
from jax.experimental import pallas as pl
from jax.experimental.pallas import tpu as pltpu
from jax.experimental.pallas import tpu_sc as plsc

_N = 10000
_E = 320000
_D = 128
_U = 128

_NC = 2
_NS = 16
_L = 16
_NW = _NC * _NS
_EPW = _E // _NW
_C = 80
_NCHUNK = _EPW // _C
_NPAD = 10240
_RPT = _NPAD // _NS
_WPT = _NPAD // _NS


def _lane_bcast(vec, j):
  idx = jnp.full((_L, 1), j, dtype=jnp.int32)
  dnums = lax.GatherDimensionNumbers(
      offset_dims=(), collapsed_slice_dims=(0,), start_index_map=(0,))
  return lax.gather(vec, idx, dnums, (1,),
                    mode=lax.GatherScatterMode.PROMISE_IN_BOUNDS)


def _sc_body(nf, ei, ew, aggp, wsump, src_v, dst_v, w_v, rows_v, agg_sh, ws_sh):
  cid = lax.axis_index("c")
  sid = lax.axis_index("s")
  wid = sid * _NC + cid
  zero = jnp.zeros((_L,), jnp.float32)

  def _zfill(i, carry):
    for k in range(_D // _L):
      rows_v[i, pl.ds(k * _L, _L)] = zero
    return carry
  lax.fori_loop(0, _C, _zfill, 0)
  for k in range(_C // _L):
    w_v[pl.ds(k * _L, _L)] = zero

  row0 = sid * _RPT
  for i in range(_RPT // _C):
    pltpu.sync_copy(rows_v, agg_sh.at[pl.ds(row0 + i * _C, _C)])
  w0 = sid * _WPT
  for i in range(_WPT // _C):
    pltpu.sync_copy(w_v, ws_sh.at[pl.ds(w0 + i * _C, _C)])
  plsc.subcore_barrier()

  e0 = wid * _EPW

  def _chunk(t, carry):
    off = e0 + t * _C
    pltpu.sync_copy(ei.at[pl.ds(_E + off, _C)], src_v)
    pltpu.sync_copy(ei.at[pl.ds(off, _C)], dst_v)
    pltpu.sync_copy(ew.at[pl.ds(off, _C)], w_v)
    pltpu.sync_copy(nf.at[src_v], rows_v)
    for g in range(_C // _L):
      w16 = w_v[pl.ds(g * _L, _L)]
      for j in range(_L):
        r = g * _L + j
        wj = _lane_bcast(w16, j)
        for k in range(_D // _L):
          rows_v[r, pl.ds(k * _L, _L)] = rows_v[r, pl.ds(k * _L, _L)] * wj
    pltpu.sync_copy(rows_v, agg_sh.at[dst_v], add=True)
    pltpu.sync_copy(w_v, ws_sh.at[dst_v], add=True)
    return carry

  lax.fori_loop(0, _NCHUNK, _chunk, 0)

  plsc.subcore_barrier()
  pltpu.sync_copy(agg_sh.at[pl.ds(row0, _RPT)],
                  aggp.at[cid, pl.ds(row0, _RPT)])
  pltpu.sync_copy(ws_sh.at[pl.ds(w0, _WPT)], wsump.at[cid, pl.ds(w0, _WPT)])


_sc_agg = functools.partial(
    pl.kernel,
    out_type=(jax.ShapeDtypeStruct((_NC, _NPAD, _D), jnp.float32),
              jax.ShapeDtypeStruct((_NC, _NPAD), jnp.float32)),
    mesh=plsc.VectorSubcoreMesh(core_axis_name="c", subcore_axis_name="s"),
    scratch_types=[
        pltpu.VMEM((_C,), jnp.int32),
        pltpu.VMEM((_C,), jnp.int32),
        pltpu.VMEM((_C,), jnp.float32),
        pltpu.VMEM((_C, _D), jnp.float32),
        pltpu.VMEM_SHARED((_NPAD, _D), jnp.float32),
        pltpu.VMEM_SHARED((_NPAD,), jnp.float32),
    ],
)(_sc_body)


def _tc_body(aggp_ref, ws_ref, w_ref, b_ref, out_ref):
  s = aggp_ref[0] + aggp_ref[1]
  ws = ws_ref[0] + ws_ref[1]
  denom = jnp.where(ws > 0.0, ws, 1.0)
  combined = s / denom
  acc = lax.dot_general(combined, w_ref[...], (((1,), (0,)), ((), ())),
                        preferred_element_type=jnp.float32,
                        precision=lax.Precision.HIGHEST)
  out_ref[...] = jnp.maximum(acc + b_ref[...], 0.0)


_BN = 400


def _tc_finish(aggp, wsum3, W, b2):
  return pl.pallas_call(
      _tc_body,
      grid=(_N // _BN,),
      in_specs=[
          pl.BlockSpec((_NC, _BN, _D), lambda i: (0, i, 0)),
          pl.BlockSpec((_NC, _BN, 1), lambda i: (0, i, 0)),
          pl.BlockSpec((_D, _U), lambda i: (0, 0)),
          pl.BlockSpec((1, _U), lambda i: (0, 0)),
      ],
      out_specs=pl.BlockSpec((_BN, _U), lambda i: (i, 0)),
      out_shape=jax.ShapeDtypeStruct((_N, _U), jnp.float32),
  )(aggp, wsum3, W, b2)


@jax.jit
def _impl(node_features, edge_index, edge_weights, W, b):
  ei_flat = edge_index.astype(jnp.int32).reshape(2 * _E)
  aggp, wsump = _sc_agg(node_features, ei_flat, edge_weights)
  return _tc_finish(aggp, wsump.reshape(_NC, _NPAD, 1), W, b.reshape(1, _U))


def kernel(node_features, edge_index, edge_weights, W, b):
  return _impl(node_features, edge_index, edge_weights, W, b)


# speedup vs baseline: 9.1167x; 1.7762x over previous
import functools

import jax
import jax.numpy as jnp
from jax import lax
from jax.experimental import pallas as pl
from jax.experimental.pallas import tpu as pltpu
from jax.experimental.pallas import tpu_sc as plsc

_N = 10000
_E = 320000
_D = 128
_U = 128

_NC = 2
_NS = 16
_L = 16
_NW = _NC * _NS
_EPW = _E // _NW
_C = 80
_CPS = 25
_NSEG = _EPW // (_C * _CPS)
_ZT = 10
_RPZ = _N // _ZT


def _lane_bcast(vec, j):
  idx = jnp.full((_L, 1), j, dtype=jnp.int32)
  dnums = lax.GatherDimensionNumbers(
      offset_dims=(), collapsed_slice_dims=(0,), start_index_map=(0,))
  return lax.gather(vec, idx, dnums, (1,),
                    mode=lax.GatherScatterMode.PROMISE_IN_BOUNDS)


def _sc_body(nf, src4, dst4, ew4, aggp, wsump,
             src_v, dst_v, w_v, zb_v, wsb_v, rows0, rows1, agg_sh, ws_sh,
             g0, g1):
  cid = lax.axis_index("c")
  sid = lax.axis_index("s")
  wid = sid * _NC + cid
  zero = jnp.zeros((_L,), jnp.float32)

  def _zfill(i, carry):
    for k in range(_D // _L):
      rows0[i, pl.ds(k * _L, _L)] = zero
    return carry
  lax.fori_loop(0, _C, _zfill, 0)
  for k in range(_C // _L):
    zb_v[pl.ds(k * _L, _L)] = zero

  @pl.when(sid < _ZT)
  def _zero_acc():
    row0 = sid * _RPZ
    for i in range(_RPZ // _C):
      pltpu.sync_copy(rows0, agg_sh.at[pl.ds(row0 + i * _C, _C)])
      pltpu.sync_copy(zb_v, ws_sh.at[pl.ds(row0 + i * _C, _C)])
    rem = _RPZ - (_RPZ // _C) * _C
    pltpu.sync_copy(rows0.at[pl.ds(0, rem)],
                    agg_sh.at[pl.ds(row0 + _RPZ - rem, rem)])
    pltpu.sync_copy(zb_v.at[pl.ds(0, rem)],
                    ws_sh.at[pl.ds(row0 + _RPZ - rem, rem)])

  plsc.subcore_barrier()

  def _gather(t, buf, sem):
    return pltpu.make_async_copy(nf.at[src_v.at[t]], buf, sem)

  def _scale(t, buf):
    for g in range(_C // _L):
      w16 = w_v[t, pl.ds(g * _L, _L)]
      for j in range(_L):
        r = g * _L + j
        wj = _lane_bcast(w16, j)
        for k in range(_D // _L):
          buf[r, pl.ds(k * _L, _L)] = buf[r, pl.ds(k * _L, _L)] * wj

  def _scatter(t, buf):
    pltpu.sync_copy(buf, agg_sh.at[dst_v.at[t]], add=True)
    pltpu.sync_copy(w_v.at[t], ws_sh.at[dst_v.at[t]], add=True)

  def _segment(seg, carry):
    pltpu.sync_copy(src4.at[wid, seg], src_v)
    pltpu.sync_copy(dst4.at[wid, seg], dst_v)
    pltpu.sync_copy(ew4.at[wid, seg], w_v)
    _gather(0, rows0, g0).start()

    def _pair(i, c):
      t0 = 2 * i
      t1 = t0 + 1
      _gather(t1, rows1, g1).start()
      _gather(t0, rows0, g0).wait()
      _scale(t0, rows0)
      _scatter(t0, rows0)
      _gather(t0 + 2, rows0, g0).start()
      _gather(t1, rows1, g1).wait()
      _scale(t1, rows1)
      _scatter(t1, rows1)
      return c

    lax.fori_loop(0, (_CPS - 1) // 2, _pair, 0)
    _gather(_CPS - 1, rows0, g0).wait()
    _scale(_CPS - 1, rows0)
    _scatter(_CPS - 1, rows0)
    return carry

  lax.fori_loop(0, _NSEG, _segment, 0)

  plsc.subcore_barrier()

  @pl.when(sid < _ZT)
  def _copy_out():
    row0 = sid * _RPZ
    pltpu.sync_copy(agg_sh.at[pl.ds(row0, _RPZ)],
                    aggp.at[cid, pl.ds(row0, _RPZ)])
    pltpu.sync_copy(ws_sh.at[pl.ds(row0, _RPZ)], wsb_v)
    pltpu.sync_copy(wsb_v, wsump.at[pl.ds(cid * _N + row0, _RPZ)])


_sc_agg = functools.partial(
    pl.kernel,
    out_type=(jax.ShapeDtypeStruct((_NC, _N, _D), jnp.float32),
              jax.ShapeDtypeStruct((_NC * _N,), jnp.float32)),
    mesh=plsc.VectorSubcoreMesh(core_axis_name="c", subcore_axis_name="s"),
    scratch_types=[
        pltpu.VMEM((_CPS, _C), jnp.int32),
        pltpu.VMEM((_CPS, _C), jnp.int32),
        pltpu.VMEM((_CPS, _C), jnp.float32),
        pltpu.VMEM((_C,), jnp.float32),
        pltpu.VMEM((_RPZ,), jnp.float32),
        pltpu.VMEM((_C, _D), jnp.float32),
        pltpu.VMEM((_C, _D), jnp.float32),
        pltpu.VMEM_SHARED((_N, _D), jnp.float32),
        pltpu.VMEM_SHARED((_N,), jnp.float32),
        pltpu.SemaphoreType.DMA,
        pltpu.SemaphoreType.DMA,
    ],
)(_sc_body)


def _tc_body(aggp_ref, ws_ref, w_ref, b_ref, out_ref):
  s = aggp_ref[0] + aggp_ref[1]
  ws = ws_ref[0] + ws_ref[1]
  denom = jnp.where(ws > 0.0, ws, 1.0)
  combined = s / denom
  acc = lax.dot_general(combined, w_ref[...], (((1,), (0,)), ((), ())),
                        preferred_element_type=jnp.float32,
                        precision=lax.Precision.HIGHEST)
  out_ref[...] = jnp.maximum(acc + b_ref[...], 0.0)


_BN = 400


def _tc_finish(aggp, wsum3, W, b2):
  return pl.pallas_call(
      _tc_body,
      grid=(_N // _BN,),
      in_specs=[
          pl.BlockSpec((_NC, _BN, _D), lambda i: (0, i, 0)),
          pl.BlockSpec((_NC, _BN, 1), lambda i: (0, i, 0)),
          pl.BlockSpec((_D, _U), lambda i: (0, 0)),
          pl.BlockSpec((1, _U), lambda i: (0, 0)),
      ],
      out_specs=pl.BlockSpec((_BN, _U), lambda i: (i, 0)),
      out_shape=jax.ShapeDtypeStruct((_N, _U), jnp.float32),
  )(aggp, wsum3, W, b2)


@jax.jit
def _impl(node_features, edge_index, edge_weights, W, b):
  ei = edge_index.astype(jnp.int32)
  src4 = ei[1].reshape(_NW, _NSEG, _CPS, _C)
  dst4 = ei[0].reshape(_NW, _NSEG, _CPS, _C)
  ew4 = edge_weights.reshape(_NW, _NSEG, _CPS, _C)
  aggp, wsump = _sc_agg(node_features, src4, dst4, ew4)
  return _tc_finish(aggp, wsump.reshape(_NC, _N, 1), W, b.reshape(1, _U))


def kernel(node_features, edge_index, edge_weights, W, b):
  return _impl(node_features, edge_index, edge_weights, W, b)
